# baseline (speedup 1.0000x reference)
import jax
import jax.numpy as jnp
from jax import lax
from jax.experimental import pallas as pl
from jax.experimental.pallas import tpu as pltpu
from jax.experimental.pallas import tpu_sc as plsc

K = 16
ROWS = 8192
COLS = 8192
L = 16
NC = 2
NS = 16
NW = NC * NS
RPW = ROWS // NW
NV = COLS // L
B = 4
NCH = RPW // B
U1 = 8
U3 = 4
MU = 1e-3

LOG2E = 1.4426950408889634
LN2 = 0.6931471805599453
MAGIC = 12582912.0
EXP2_C = (1.0, 0.6931472, 0.24022211, 0.055503406, 0.009670768, 0.0013395286)
LOG2_C = (0.58496267, 0.96179616, -0.32062477, 0.14251883, -0.07068618,
          0.03753272, -0.024825774, 0.014598605)


def _fast_exp(x):
    t = jnp.maximum(x * LOG2E, -126.0)
    nf = (t + MAGIC) - MAGIC
    r = t - nf
    p = jnp.full((L,), EXP2_C[5], jnp.float32)
    for c in EXP2_C[4::-1]:
        p = p * r + jnp.float32(c)
    ni = nf.astype(jnp.int32)
    sf = lax.bitcast_convert_type((ni + 127) << 23, jnp.float32)
    return p * sf


def _fast_log(zv):
    zb = lax.bitcast_convert_type(zv, jnp.int32)
    ex = ((zb >> 23) - 127).astype(jnp.float32)
    mant = lax.bitcast_convert_type(
        (zb & jnp.int32(0x007FFFFF)) | jnp.int32(0x3F800000), jnp.float32)
    d = mant - 1.5
    p = jnp.full((L,), LOG2_C[7], jnp.float32)
    for c in LOG2_C[6::-1]:
        p = p * d + jnp.float32(c)
    return (ex + p) * LN2


def _row_compute(buf, rb, ys, ltvec):
    def p1(i, accs):
        mxs, mns = accs
        b = rb + i * (L * U1)
        vals = [buf[pl.ds(b + j * L, L)] for j in range(U1)]
        return (tuple(jnp.maximum(a, v) for a, v in zip(mxs, vals)),
                tuple(jnp.minimum(a, v) for a, v in zip(mns, vals)))
    mxs, mns = lax.fori_loop(
        0, NV // U1, p1,
        ((jnp.full((L,), -jnp.inf, jnp.float32),) * U1,
         (jnp.full((L,), jnp.inf, jnp.float32),) * U1))
    mx, mn = mxs[0], mns[0]
    for a in mxs[1:]:
        mx = jnp.maximum(mx, a)
    for a in mns[1:]:
        mn = jnp.minimum(mn, a)
    m = jnp.max(mx)
    smin = jnp.min(mn)

    def p2(i, zs):
        b = rb + i * (L * U1)
        return tuple(zacc + _fast_exp(buf[pl.ds(b + j * L, L)] - m)
                     for j, zacc in enumerate(zs))
    zs = (jnp.zeros((L,), jnp.float32),) * U1
    zv = zs[0]
    for a in zs[1:]:
        zv = zv + a
    z = jnp.float32(1.0)

    logz = _fast_log(jnp.full((L,), z))[0]
    uvec = ltvec + (m + logz)
    khi = jnp.sum(jnp.where(uvec < m + MU, 1, 0))
    klo = jnp.sum(jnp.where(uvec < smin - MU, 1, 0))

    def fast_fill():
        yc = ys[0]
        for k in range(1, K):
            yc = jnp.where(klo >= k, ys[k], yc)
        ocv = (jnp.full((L,), yc) /
               jnp.maximum(jnp.full((L,), jnp.float32(COLS) * yc), 1e-30))

        def pf(i, c):
            b = rb + i * (L * U1)
            for j in range(U1):
                buf[pl.ds(b + j * L, L)] = ocv
            return c
        lax.fori_loop(0, NV // U1, pf, 0)

    def slow_path():
        us = [uvec[k] for k in range(K - 1)]

        def p3(i, ds_):
            b = rb + i * (L * U3)
            out = []
            for j, dacc in enumerate(ds_):
                s = buf[pl.ds(b + j * L, L)]
                yq = jnp.full((L,), ys[0])
                for k in range(K - 1):
                    yq = jnp.where(s > us[k], ys[k + 1], yq)
                buf[pl.ds(b + j * L, L)] = yq
                out.append(dacc + yq)
            return tuple(out)
        ds_ = lax.fori_loop(0, NV // U3, p3,
                            (jnp.zeros((L,), jnp.float32),) * U3)
        dv = ds_[0]
        for a in ds_[1:]:
            dv = dv + a
        denom = jnp.maximum(jnp.sum(dv), 1e-30)
        rdv = jnp.ones((L,), jnp.float32) / denom

        def p4(i, c):
            b = rb + i * (L * U1)
            for j in range(U1):
                buf[pl.ds(b + j * L, L)] = buf[pl.ds(b + j * L, L)] * rdv
            return c
        lax.fori_loop(0, NV // U1, p4, 0)

    pl.when(khi == klo)(fast_fill)
    pl.when(khi != klo)(slow_path)


def _sc_body(lt_hbm, y_hbm, s_hbm, o_hbm, buf0, buf1, ltv, yv,
             isem0, isem1, osem0, osem1):
    wid = lax.axis_index("s") * NC + lax.axis_index("c")
    base = wid * RPW

    pltpu.sync_copy(lt_hbm, ltv)
    pltpu.sync_copy(y_hbm, yv)
    ltvec = ltv[...]
    yvec = yv[...]
    ys = [yvec[k] for k in range(K)]

    def in_start(c, buf, isem):
        w0 = (base + c * B) * COLS
        pltpu.make_async_copy(s_hbm.at[pl.ds(w0, B * COLS)], buf, isem).start()

    def in_wait(buf, isem):
        pltpu.make_async_copy(s_hbm.at[pl.ds(base * COLS, B * COLS)], buf,
                              isem).wait()

    def out_wait(buf, osem):
        pltpu.make_async_copy(buf, o_hbm.at[pl.ds(base * COLS, B * COLS)],
                              osem).wait()

    def chunk_compute(c, buf, osem):
        w0 = (base + c * B) * COLS

        def rowfn(r, carry):
            rb = r * COLS
            _row_compute(buf, rb, ys, ltvec)
            pltpu.make_async_copy(buf.at[pl.ds(rb, COLS)],
                                  o_hbm.at[pl.ds(w0 + rb, COLS)], osem).start()
            return carry
        lax.fori_loop(0, B, rowfn, 0)

    in_start(0, buf0, isem0)
    in_start(1, buf1, isem1)
    in_wait(buf0, isem0)
    chunk_compute(0, buf0, osem0)
    out_wait(buf0, osem0)
    in_start(2, buf0, isem0)
    in_wait(buf1, isem1)
    chunk_compute(1, buf1, osem1)

    def loop(gp, carry):
        ca = 2 * gp
        out_wait(buf1, osem1)
        in_start(ca + 1, buf1, isem1)
        in_wait(buf0, isem0)
        chunk_compute(ca, buf0, osem0)
        out_wait(buf0, osem0)
        in_start(jnp.minimum(ca + 2, NCH - 1), buf0, isem0)
        in_wait(buf1, isem1)
        chunk_compute(ca + 1, buf1, osem1)
        return carry
    lax.fori_loop(1, NCH // 2, loop, 0)

    in_wait(buf0, isem0)
    out_wait(buf1, osem1)


def kernel(scores, thresholds, y):
    orig_shape = scores.shape
    s2 = scores.reshape(ROWS * COLS)
    logt = jnp.pad(jnp.log(thresholds), (0, 1), constant_values=jnp.inf)
    mesh = plsc.VectorSubcoreMesh(core_axis_name="c", subcore_axis_name="s")
    out = pl.kernel(
        _sc_body,
        out_type=jax.ShapeDtypeStruct((ROWS * COLS,), jnp.float32),
        mesh=mesh,
        scratch_types=[
            pltpu.VMEM((B * COLS,), jnp.float32),
            pltpu.VMEM((B * COLS,), jnp.float32),
            pltpu.VMEM((L,), jnp.float32),
            pltpu.VMEM((L,), jnp.float32),
            pltpu.SemaphoreType.DMA,
            pltpu.SemaphoreType.DMA,
            pltpu.SemaphoreType.DMA,
            pltpu.SemaphoreType.DMA,
        ],
        compiler_params=pltpu.CompilerParams(needs_layout_passes=False),
    )(logt, y, s2)
    return out.reshape(orig_shape)


# speedup vs baseline: 1.8249x; 1.8249x over previous
import jax
import jax.numpy as jnp
from jax import lax
from jax.experimental import pallas as pl
from jax.experimental.pallas import tpu as pltpu
from jax.experimental.pallas import tpu_sc as plsc

K = 16
ROWS = 8192
COLS = 8192
L = 16
NC = 2
NS = 16
NW = NC * NS
RPW = ROWS // NW
NV = COLS // L
B = 4
NCH = RPW // B
U1 = 8
U3 = 4
MU = 1e-3

LOG2E = 1.4426950408889634
LN2 = 0.6931471805599453
MAGIC = 12582912.0
EXP2_C = (1.0, 0.6931472, 0.24022211, 0.055503406, 0.009670768, 0.0013395286)
LOG2_C = (0.58496267, 0.96179616, -0.32062477, 0.14251883, -0.07068618,
          0.03753272, -0.024825774, 0.014598605)


def _fast_exp(x):
    t = jnp.maximum(x * LOG2E, -126.0)
    nf = (t + MAGIC) - MAGIC
    r = t - nf
    p = jnp.full((L,), EXP2_C[5], jnp.float32)
    for c in EXP2_C[4::-1]:
        p = p * r + jnp.float32(c)
    ni = nf.astype(jnp.int32)
    sf = lax.bitcast_convert_type((ni + 127) << 23, jnp.float32)
    return p * sf


def _fast_log(zv):
    zb = lax.bitcast_convert_type(zv, jnp.int32)
    ex = ((zb >> 23) - 127).astype(jnp.float32)
    mant = lax.bitcast_convert_type(
        (zb & jnp.int32(0x007FFFFF)) | jnp.int32(0x3F800000), jnp.float32)
    d = mant - 1.5
    p = jnp.full((L,), LOG2_C[7], jnp.float32)
    for c in LOG2_C[6::-1]:
        p = p * d + jnp.float32(c)
    return (ex + p) * LN2


def _row_compute(buf, rb, ys, ltvec):
    @plsc.parallel_loop(0, NV // U1, carry=(
        (jnp.full((L,), -jnp.inf, jnp.float32),) * U1,
        (jnp.full((L,), jnp.inf, jnp.float32),) * U1))
    def p1(i, accs):
        mxs, mns = accs
        b = rb + i * (L * U1)
        vals = [buf[pl.ds(b + j * L, L)] for j in range(U1)]
        return (tuple(jnp.maximum(a, v) for a, v in zip(mxs, vals)),
                tuple(jnp.minimum(a, v) for a, v in zip(mns, vals)))
    mxs, mns = p1
    mx, mn = mxs[0], mns[0]
    for a in mxs[1:]:
        mx = jnp.maximum(mx, a)
    for a in mns[1:]:
        mn = jnp.minimum(mn, a)
    m = jnp.max(mx)
    smin = jnp.min(mn)

    @plsc.parallel_loop(0, NV // U1,
                        carry=(jnp.zeros((L,), jnp.float32),) * U1)
    def p2(i, zs):
        b = rb + i * (L * U1)
        return tuple(zacc + _fast_exp(buf[pl.ds(b + j * L, L)] - m)
                     for j, zacc in enumerate(zs))
    zs = p2
    zv = zs[0]
    for a in zs[1:]:
        zv = zv + a
    z = jnp.sum(zv)

    logz = _fast_log(jnp.full((L,), z))[0]
    uvec = ltvec + (m + logz)
    khi = jnp.sum(jnp.where(uvec < m + MU, 1, 0))
    klo = jnp.sum(jnp.where(uvec < smin - MU, 1, 0))

    def fast_fill():
        yc = ys[0]
        for k in range(1, K):
            yc = jnp.where(klo >= k, ys[k], yc)
        ocv = (jnp.full((L,), yc) /
               jnp.maximum(jnp.full((L,), jnp.float32(COLS) * yc), 1e-30))

        @plsc.parallel_loop(0, NV // U1)
        def pf(i):
            b = rb + i * (L * U1)
            for j in range(U1):
                buf[pl.ds(b + j * L, L)] = ocv

    def slow_path():
        us = [uvec[k] for k in range(K - 1)]

        @plsc.parallel_loop(0, NV // U3,
                            carry=(jnp.zeros((L,), jnp.float32),) * U3)
        def p3(i, ds_):
            b = rb + i * (L * U3)
            out = []
            for j, dacc in enumerate(ds_):
                s = buf[pl.ds(b + j * L, L)]
                yq = jnp.full((L,), ys[0])
                for k in range(K - 1):
                    yq = jnp.where(s > us[k], ys[k + 1], yq)
                buf[pl.ds(b + j * L, L)] = yq
                out.append(dacc + yq)
            return tuple(out)
        ds_ = p3
        dv = ds_[0]
        for a in ds_[1:]:
            dv = dv + a
        denom = jnp.maximum(jnp.sum(dv), 1e-30)
        rdv = jnp.ones((L,), jnp.float32) / denom

        @plsc.parallel_loop(0, NV // U1)
        def p4(i):
            b = rb + i * (L * U1)
            for j in range(U1):
                buf[pl.ds(b + j * L, L)] = buf[pl.ds(b + j * L, L)] * rdv

    pl.when(khi == klo)(fast_fill)
    pl.when(khi != klo)(slow_path)


def _sc_body(lt_hbm, y_hbm, s_hbm, o_hbm, buf0, buf1, ltv, yv,
             isem0, isem1, osem0, osem1):
    wid = lax.axis_index("s") * NC + lax.axis_index("c")
    base = wid * RPW

    pltpu.sync_copy(lt_hbm, ltv)
    pltpu.sync_copy(y_hbm, yv)
    ltvec = ltv[...]
    yvec = yv[...]
    ys = [yvec[k] for k in range(K)]

    def in_start(c, buf, isem):
        w0 = (base + c * B) * COLS
        pltpu.make_async_copy(s_hbm.at[pl.ds(w0, B * COLS)], buf, isem).start()

    def in_wait(buf, isem):
        pltpu.make_async_copy(s_hbm.at[pl.ds(base * COLS, B * COLS)], buf,
                              isem).wait()

    def out_wait(buf, osem):
        pltpu.make_async_copy(buf, o_hbm.at[pl.ds(base * COLS, B * COLS)],
                              osem).wait()

    def chunk_compute(c, buf, osem):
        w0 = (base + c * B) * COLS

        def rowfn(r, carry):
            rb = r * COLS
            _row_compute(buf, rb, ys, ltvec)
            pltpu.make_async_copy(buf.at[pl.ds(rb, COLS)],
                                  o_hbm.at[pl.ds(w0 + rb, COLS)], osem).start()
            return carry
        lax.fori_loop(0, B, rowfn, 0)

    in_start(0, buf0, isem0)
    in_start(1, buf1, isem1)
    in_wait(buf0, isem0)
    chunk_compute(0, buf0, osem0)
    out_wait(buf0, osem0)
    in_start(2, buf0, isem0)
    in_wait(buf1, isem1)
    chunk_compute(1, buf1, osem1)

    def loop(gp, carry):
        ca = 2 * gp
        out_wait(buf1, osem1)
        in_start(ca + 1, buf1, isem1)
        in_wait(buf0, isem0)
        chunk_compute(ca, buf0, osem0)
        out_wait(buf0, osem0)
        in_start(jnp.minimum(ca + 2, NCH - 1), buf0, isem0)
        in_wait(buf1, isem1)
        chunk_compute(ca + 1, buf1, osem1)
        return carry
    lax.fori_loop(1, NCH // 2, loop, 0)

    in_wait(buf0, isem0)
    out_wait(buf1, osem1)


def kernel(scores, thresholds, y):
    orig_shape = scores.shape
    s2 = scores.reshape(ROWS * COLS)
    logt = jnp.pad(jnp.log(thresholds), (0, 1), constant_values=jnp.inf)
    mesh = plsc.VectorSubcoreMesh(core_axis_name="c", subcore_axis_name="s")
    out = pl.kernel(
        _sc_body,
        out_type=jax.ShapeDtypeStruct((ROWS * COLS,), jnp.float32),
        mesh=mesh,
        scratch_types=[
            pltpu.VMEM((B * COLS,), jnp.float32),
            pltpu.VMEM((B * COLS,), jnp.float32),
            pltpu.VMEM((L,), jnp.float32),
            pltpu.VMEM((L,), jnp.float32),
            pltpu.SemaphoreType.DMA,
            pltpu.SemaphoreType.DMA,
            pltpu.SemaphoreType.DMA,
            pltpu.SemaphoreType.DMA,
        ],
        compiler_params=pltpu.CompilerParams(needs_layout_passes=False),
    )(logt, y, s2)
    return out.reshape(orig_shape)
